# baseline (speedup 1.0000x reference)
import jax
import jax.numpy as jnp
from jax.experimental import pallas as pl
from jax.experimental.pallas import tpu as pltpu

EPS = 1e-7
MAX_NORM = 1.0 - 1e-5


def _fused_body(x_ref, adj_ref, o_ref, xt_ref):
    @pl.when(pl.program_id(0) == 0)
    def _():
        xx = x_ref[...]
        norm = jnp.clip(jnp.sqrt(jnp.sum(xx * xx, axis=-1, keepdims=True)), EPS, None)
        z = jnp.clip(norm, None, MAX_NORM)
        atanh = 0.5 * jnp.log((1.0 + z) / (1.0 - z))
        xt = atanh * xx / norm
        xt_hi = xt.astype(jnp.bfloat16)
        xt_ref[0] = xt_hi
        xt_ref[1] = (xt - xt_hi.astype(jnp.float32)).astype(jnp.bfloat16)

    blk = adj_ref[...]
    blk_bf = blk.astype(jnp.bfloat16)
    acc = (jnp.dot(blk_bf, xt_ref[0], preferred_element_type=jnp.float32)
           + jnp.dot(blk_bf, xt_ref[1], preferred_element_type=jnp.float32))
    cnt = jnp.sum(blk, axis=1, keepdims=True)
    agg = acc / cnt
    norm = jnp.clip(jnp.sqrt(jnp.sum(agg * agg, axis=-1, keepdims=True)), EPS, None)
    res = jnp.tanh(norm) * agg / norm
    norm2 = jnp.clip(jnp.sqrt(jnp.sum(res * res, axis=-1, keepdims=True)), EPS, None)
    o_ref[...] = res * jnp.minimum(1.0, MAX_NORM / norm2)


def kernel(x, adj):
    N, D = x.shape
    BM = 400
    return pl.pallas_call(
        _fused_body,
        grid=(N // BM,),
        in_specs=[
            pl.BlockSpec((N, D), lambda i: (0, 0)),
            pl.BlockSpec((BM, N), lambda i: (i, 0)),
        ],
        out_specs=pl.BlockSpec((BM, D), lambda i: (i, 0)),
        out_shape=jax.ShapeDtypeStruct((N, D), jnp.float32),
        scratch_shapes=[pltpu.VMEM((2, N, D), jnp.bfloat16)],
        compiler_params=pltpu.CompilerParams(
            dimension_semantics=("arbitrary",),
        ),
    )(x, adj)


# speedup vs baseline: 1.1078x; 1.1078x over previous
import jax
import jax.numpy as jnp
from jax.experimental import pallas as pl
from jax.experimental.pallas import tpu as pltpu

EPS = 1e-7
MAX_NORM = 1.0 - 1e-5


def _fused_body(x_ref, adj_ref, o_ref, xt_ref):
    @pl.when(pl.program_id(0) == 0)
    def _():
        xx = x_ref[...]
        norm = jnp.clip(jnp.sqrt(jnp.sum(xx * xx, axis=-1, keepdims=True)), EPS, None)
        z = jnp.clip(norm, None, MAX_NORM)
        atanh = 0.5 * jnp.log((1.0 + z) / (1.0 - z))
        xt_ref[...] = (atanh * xx / norm).astype(jnp.bfloat16)

    blk = adj_ref[...]
    acc = jnp.dot(blk.astype(jnp.bfloat16), xt_ref[...],
                  preferred_element_type=jnp.float32)
    cnt = jnp.sum(blk, axis=1, keepdims=True)
    agg = acc / cnt
    norm = jnp.clip(jnp.sqrt(jnp.sum(agg * agg, axis=-1, keepdims=True)), EPS, None)
    res = jnp.tanh(norm) * agg / norm
    norm2 = jnp.clip(jnp.sqrt(jnp.sum(res * res, axis=-1, keepdims=True)), EPS, None)
    o_ref[...] = res * jnp.minimum(1.0, MAX_NORM / norm2)


def kernel(x, adj):
    N, D = x.shape
    BM = 400
    return pl.pallas_call(
        _fused_body,
        grid=(N // BM,),
        in_specs=[
            pl.BlockSpec((N, D), lambda i: (0, 0)),
            pl.BlockSpec((BM, N), lambda i: (i, 0)),
        ],
        out_specs=pl.BlockSpec((BM, D), lambda i: (i, 0)),
        out_shape=jax.ShapeDtypeStruct((N, D), jnp.float32),
        scratch_shapes=[pltpu.VMEM((N, D), jnp.bfloat16)],
        compiler_params=pltpu.CompilerParams(
            dimension_semantics=("arbitrary",),
        ),
    )(x, adj)
